# baseline (speedup 1.0000x reference)
import jax
import jax.numpy as jnp
from jax.experimental import pallas as pl
from jax.experimental.pallas import tpu as pltpu


def _dense_layer(x, m, inv_cnt, w_ref, a_ref):
    att = a_ref[...].astype(jnp.bfloat16)
    H = w_ref.shape[1]
    N = x.shape[0]
    h = jax.lax.dot_general(x.astype(jnp.bfloat16), w_ref[...].astype(jnp.bfloat16),
                            (((1,), (0,)), ((), ())),
                            preferred_element_type=jnp.float32)
    hb = h.astype(jnp.bfloat16)
    a_dst = jax.lax.dot_general(att[:H], hb, (((0,), (1,)), ((), ())),
                                preferred_element_type=jnp.float32
                                ).astype(jnp.bfloat16)
    a_src = jax.lax.dot_general(hb, att[H:], (((1,), (0,)), ((), ())),
                                preferred_element_type=jnp.float32
                                ).astype(jnp.bfloat16)
    A = a_src + a_dst
    A = jnp.maximum(A, jnp.bfloat16(0.2) * A)
    P = m * jnp.exp(A)
    haug = jnp.concatenate([hb, jnp.ones((N, 1), jnp.bfloat16)], axis=1)
    saug = jax.lax.dot_general(P, haug, (((0,), (0,)), ((), ())),
                               preferred_element_type=jnp.float32)
    x = saug[:, :H] * (1.0 / (saug[:, H:] + 1e-16) * inv_cnt)
    return jnp.where(x > 0.0, x, jnp.exp(x) - 1.0)


def _gat_kernel(adj_ref, tf_ref, w1_ref, a1_ref, w2_ref, a2_ref, w3_ref,
                a3_ref, out_ref, m_sc, hb_sc, x_sc):
    i = pl.program_id(0)
    B, N, NT = adj_ref.shape
    H = w1_ref.shape[1]

    @pl.when(i == 0)
    def _init():
        x0 = tf_ref[0]
        h1 = jax.lax.dot_general(x0.astype(jnp.bfloat16),
                                 w1_ref[...].astype(jnp.bfloat16),
                                 (((1,), (0,)), ((), ())),
                                 preferred_element_type=jnp.float32)
        hb_sc[...] = h1.astype(jnp.bfloat16)

    @pl.when(i < 4)
    def _layer1_tile():
        m_i = adj_ref[0]
        for b in range(1, B):
            m_i = m_i + adj_ref[b]
        mt = m_i.astype(jnp.bfloat16)
        m_sc[:, pl.ds(i * NT, NT)] = mt

        att = a1_ref[...].astype(jnp.bfloat16)
        hb = hb_sc[...]
        a_src = jax.lax.dot_general(hb, att[H:], (((1,), (0,)), ((), ())),
                                    preferred_element_type=jnp.float32
                                    ).astype(jnp.bfloat16)
        htile = hb_sc[pl.ds(i * NT, NT), :]
        a_dst = jax.lax.dot_general(att[:H], htile, (((0,), (1,)), ((), ())),
                                    preferred_element_type=jnp.float32
                                    ).astype(jnp.bfloat16)
        A = a_src + a_dst
        A = jnp.maximum(A, jnp.bfloat16(0.2) * A)
        P = mt * jnp.exp(A)
        ones_col = jnp.ones((N, 1), jnp.bfloat16)
        haug = jnp.concatenate([hb, ones_col], axis=1)
        saug = jax.lax.dot_general(P, haug, (((0,), (0,)), ((), ())),
                                   preferred_element_type=jnp.float32)
        cnt = jax.lax.dot_general(mt, ones_col, (((0,), (0,)), ((), ())),
                                  preferred_element_type=jnp.float32)
        xt = saug[:, :H] * (1.0 / (saug[:, H:] + 1e-16)
                            / jnp.maximum(cnt, 1.0))
        x_sc[pl.ds(i * NT, NT), :] = jnp.where(xt > 0.0, xt, jnp.exp(xt) - 1.0)

    @pl.when(i == 4)
    def _layers_2_3():
        m = m_sc[...]
        cnt = jax.lax.dot_general(m, jnp.ones((N, 1), jnp.bfloat16),
                                  (((0,), (0,)), ((), ())),
                                  preferred_element_type=jnp.float32)
        inv_cnt = 1.0 / jnp.maximum(cnt, 1.0)
        x = _dense_layer(x_sc[...], m, inv_cnt, w2_ref, a2_ref)
        x = _dense_layer(x, m, inv_cnt, w3_ref, a3_ref)
        out_ref[...] = x


def kernel(sampled_edge_indices, temporal_features, W1, att1, W2, att2, W3, att3):
    B, N, D = temporal_features.shape
    O = W3.shape[1]
    NT = 128
    out = pl.pallas_call(
        _gat_kernel,
        out_shape=jax.ShapeDtypeStruct((N, O), jnp.float32),
        grid=(5,),
        in_specs=[
            pl.BlockSpec((B, N, NT), lambda i: (0, 0, jnp.minimum(i, 3))),
            pl.BlockSpec((1, N, D), lambda i: (0, 0, 0)),
            pl.BlockSpec(W1.shape, lambda i: (0, 0)),
            pl.BlockSpec(att1.shape, lambda i: (0, 0)),
            pl.BlockSpec(W2.shape, lambda i: (0, 0)),
            pl.BlockSpec(att2.shape, lambda i: (0, 0)),
            pl.BlockSpec(W3.shape, lambda i: (0, 0)),
            pl.BlockSpec(att3.shape, lambda i: (0, 0)),
        ],
        out_specs=pl.BlockSpec((N, O), lambda i: (0, 0)),
        scratch_shapes=[
            pltpu.VMEM((N, N), jnp.bfloat16),
            pltpu.VMEM((N, 64), jnp.bfloat16),
            pltpu.VMEM((N, 64), jnp.float32),
        ],
        compiler_params=pltpu.CompilerParams(
            dimension_semantics=("arbitrary",)),
    )(sampled_edge_indices, temporal_features, W1, att1, W2, att2, W3, att3)
    full = jnp.zeros((B, N, O), jnp.float32)
    return full.at[0].set(out)


# speedup vs baseline: 1.6935x; 1.6935x over previous
import jax
import jax.numpy as jnp
from jax.experimental import pallas as pl
from jax.experimental.pallas import tpu as pltpu


def _gat_kernel(adj_ref, x_ref, w1_ref, a1_ref, w2_ref, a2_ref, w3_ref,
                a3_ref, out_ref):
    B = adj_ref.shape[0]
    N = adj_ref.shape[1]
    m_i = adj_ref[0]
    for b in range(1, B):
        m_i = m_i + adj_ref[b]
    m = m_i.astype(jnp.bfloat16)
    ones_col = jnp.ones((N, 1), jnp.bfloat16)
    cnt = jax.lax.dot_general(m, ones_col, (((0,), (0,)), ((), ())),
                              preferred_element_type=jnp.float32)
    inv_cnt = 1.0 / jnp.maximum(cnt, 1.0)

    x = x_ref[...]
    for w_ref, a_ref in ((w1_ref, a1_ref), (w2_ref, a2_ref), (w3_ref, a3_ref)):
        att = a_ref[...].astype(jnp.bfloat16)
        H = w_ref.shape[1]
        h = jax.lax.dot_general(x.astype(jnp.bfloat16), w_ref[...].astype(jnp.bfloat16),
                                (((1,), (0,)), ((), ())),
                                preferred_element_type=jnp.float32)
        hb = h.astype(jnp.bfloat16)
        a_dst = jax.lax.dot_general(att[:H], hb, (((0,), (1,)), ((), ())),
                                    preferred_element_type=jnp.float32
                                    ).astype(jnp.bfloat16)
        a_src = jax.lax.dot_general(hb, att[H:], (((1,), (0,)), ((), ())),
                                    preferred_element_type=jnp.float32
                                    ).astype(jnp.bfloat16)
        A = a_src + a_dst
        A = jnp.maximum(A, jnp.bfloat16(0.2) * A)
        P = m * jnp.exp(A)
        haug = jnp.concatenate([hb, ones_col], axis=1)
        saug = jax.lax.dot_general(P, haug, (((0,), (0,)), ((), ())),
                                   preferred_element_type=jnp.float32)
        denom = saug[:, H:]
        x = saug[:, :H] * (1.0 / (denom + 1e-16) * inv_cnt)
        x = jnp.where(x > 0.0, x, jnp.exp(x) - 1.0)
    out_ref[...] = x


def kernel(sampled_edge_indices, temporal_features, W1, att1, W2, att2, W3, att3):
    B, N, D = temporal_features.shape
    O = W3.shape[1]
    x0 = temporal_features[0]
    out = pl.pallas_call(
        _gat_kernel,
        out_shape=jax.ShapeDtypeStruct((N, O), jnp.float32),
        compiler_params=pltpu.CompilerParams(
            allow_input_fusion=[False, True, False, False, False, False,
                                False, False]),
    )(sampled_edge_indices, x0, W1, att1, W2, att2, W3, att3)
    full = jnp.zeros((B, N, O), jnp.float32)
    return full.at[0].set(out)
